# baseline (speedup 1.0000x reference)
import jax
import jax.numpy as jnp
from jax.experimental import pallas as pl
from jax.experimental.pallas import tpu as pltpu

_NBUF = 4
_BM = 200


def _gcn_kernel(seq_ref, wt_ref, b_ref, adj_ref, out_ref,
                fts_ref, buf_ref, obuf_ref, sem_ref, semb_ref, osem_ref):
    n = seq_ref.shape[0]
    nsteps = n // _BM

    hm = 104

    def copy_a(k, slot):
        return pltpu.make_async_copy(
            adj_ref.at[pl.ds(k * _BM, hm), :],
            buf_ref.at[slot, pl.ds(0, hm), :],
            sem_ref.at[slot],
        )

    def copy_b(k, slot):
        return pltpu.make_async_copy(
            adj_ref.at[pl.ds(k * _BM + hm, _BM - hm), :],
            buf_ref.at[slot, pl.ds(hm, _BM - hm), :],
            semb_ref.at[slot],
        )

    def ocopy(k, slot):
        return pltpu.make_async_copy(
            obuf_ref.at[slot],
            out_ref.at[pl.ds(k * _BM, _BM), :],
            osem_ref.at[slot],
        )

    for k in range(_NBUF - 1):
        copy_a(k, k).start()
        copy_b(k, k).start()

    fc = 500
    for c in range(n // fc):
        fts = jnp.dot(seq_ref[pl.ds(c * fc, fc), :], wt_ref[...],
                      preferred_element_type=jnp.float32,
                      precision=jax.lax.Precision.HIGHEST)
        fts_ref[pl.ds(c * fc, fc), :] = fts.astype(jnp.bfloat16)

    bias = b_ref[...]

    def step(k, carry):
        slot = jax.lax.rem(k, _NBUF)
        oslot = jax.lax.rem(k, 2)
        copy_a(k, slot).wait()
        copy_b(k, slot).wait()

        @pl.when(k + _NBUF - 1 < nsteps)
        def _():
            nslot = jax.lax.rem(k + _NBUF - 1, _NBUF)
            copy_a(k + _NBUF - 1, nslot).start()
            copy_b(k + _NBUF - 1, nslot).start()

        acc = jax.lax.dot_general(
            buf_ref[slot], fts_ref[...], (((1,), (0,)), ((), ())),
            precision=jax.lax.Precision.DEFAULT,
            preferred_element_type=jnp.float32)

        @pl.when(k >= 2)
        def _():
            ocopy(k - 2, oslot).wait()

        obuf_ref[oslot] = acc + bias
        ocopy(k, oslot).start()
        return carry

    jax.lax.fori_loop(0, nsteps, step, 0)

    ocopy(nsteps - 2, (nsteps - 2) % 2).wait()
    ocopy(nsteps - 1, (nsteps - 1) % 2).wait()


def kernel(seq, adj, W, b):
    batch, n, in_ft = seq.shape
    out_ft = W.shape[0]
    seq2 = seq.reshape(batch * n, in_ft)
    adj2 = adj.reshape(batch * n, n)
    wt = W.T
    b2 = b.reshape(1, out_ft)

    out = pl.pallas_call(
        _gcn_kernel,
        in_specs=[
            pl.BlockSpec((n, in_ft), lambda: (0, 0)),
            pl.BlockSpec((in_ft, out_ft), lambda: (0, 0)),
            pl.BlockSpec((1, out_ft), lambda: (0, 0)),
            pl.BlockSpec(memory_space=pl.ANY),
        ],
        out_specs=pl.BlockSpec(memory_space=pl.ANY),
        out_shape=jax.ShapeDtypeStruct((n, out_ft), jnp.float32),
        scratch_shapes=[
            pltpu.VMEM((n, out_ft), jnp.bfloat16),
            pltpu.VMEM((_NBUF, _BM, n), jnp.float32),
            pltpu.VMEM((2, _BM, out_ft), jnp.float32),
            pltpu.SemaphoreType.DMA((_NBUF,)),
            pltpu.SemaphoreType.DMA((_NBUF,)),
            pltpu.SemaphoreType.DMA((2,)),
        ],
    )(seq2, wt, b2, adj2)

    return out.reshape(batch, n, out_ft)


# speedup vs baseline: 1.0079x; 1.0079x over previous
import jax
import jax.numpy as jnp
from jax.experimental import pallas as pl
from jax.experimental.pallas import tpu as pltpu

_NBUF = 4
_BM = 200


def _gcn_kernel(seq_ref, wt_ref, b_ref, adj_ref, out_ref,
                fts_ref, buf_ref, obuf_ref, sem_ref, osem_ref):
    n = seq_ref.shape[0]
    nsteps = n // _BM

    def copy(k, slot):
        return pltpu.make_async_copy(
            adj_ref.at[pl.ds(k * _BM, _BM), :],
            buf_ref.at[slot],
            sem_ref.at[slot],
        )

    def ocopy(k, slot):
        return pltpu.make_async_copy(
            obuf_ref.at[slot],
            out_ref.at[pl.ds(k * _BM, _BM), :],
            osem_ref.at[slot],
        )

    for k in range(_NBUF - 1):
        copy(k, k).start()

    fc = 2000
    for c in range(n // fc):
        fts_ref[pl.ds(c * fc, fc), :] = jnp.dot(
            seq_ref[pl.ds(c * fc, fc), :], wt_ref[...],
            preferred_element_type=jnp.float32,
            precision=jax.lax.Precision.DEFAULT).astype(jnp.bfloat16)

    bias = b_ref[...]

    def step(k, carry):
        slot = jax.lax.rem(k, _NBUF)
        oslot = jax.lax.rem(k, 2)
        copy(k, slot).wait()

        @pl.when(k + _NBUF - 1 < nsteps)
        def _():
            copy(k + _NBUF - 1, jax.lax.rem(k + _NBUF - 1, _NBUF)).start()

        acc = jax.lax.dot_general(
            buf_ref[slot], fts_ref[...], (((1,), (0,)), ((), ())),
            precision=jax.lax.Precision.DEFAULT,
            preferred_element_type=jnp.float32)

        @pl.when(k >= 2)
        def _():
            ocopy(k - 2, oslot).wait()

        obuf_ref[oslot] = acc + bias
        ocopy(k, oslot).start()
        return carry

    jax.lax.fori_loop(0, nsteps, step, 0)

    ocopy(nsteps - 2, (nsteps - 2) % 2).wait()
    ocopy(nsteps - 1, (nsteps - 1) % 2).wait()


def kernel(seq, adj, W, b):
    batch, n, in_ft = seq.shape
    out_ft = W.shape[0]
    seq2 = seq.reshape(batch * n, in_ft)
    adj2 = adj.reshape(batch * n, n)
    wt = W.T
    b2 = b.reshape(1, out_ft)

    out = pl.pallas_call(
        _gcn_kernel,
        in_specs=[
            pl.BlockSpec((n, in_ft), lambda: (0, 0)),
            pl.BlockSpec((in_ft, out_ft), lambda: (0, 0)),
            pl.BlockSpec((1, out_ft), lambda: (0, 0)),
            pl.BlockSpec(memory_space=pl.ANY),
        ],
        out_specs=pl.BlockSpec(memory_space=pl.ANY),
        out_shape=jax.ShapeDtypeStruct((n, out_ft), jnp.float32),
        scratch_shapes=[
            pltpu.VMEM((n, out_ft), jnp.bfloat16),
            pltpu.VMEM((_NBUF, _BM, n), jnp.float32),
            pltpu.VMEM((2, _BM, out_ft), jnp.float32),
            pltpu.SemaphoreType.DMA((_NBUF,)),
            pltpu.SemaphoreType.DMA((2,)),
        ],
    )(seq2, wt, b2, adj2)

    return out.reshape(batch, n, out_ft)
